# baseline (speedup 1.0000x reference)
import functools

import jax
import jax.numpy as jnp
from jax import lax
from jax.experimental import pallas as pl
from jax.experimental.pallas import tpu as pltpu
from jax.experimental.pallas import tpu_sc as plsc

BATCH = 16384
DIM = 64
LANES = 16
NUM_CORES = 2
NUM_SUBCORES = 16
NUM_WORKERS = NUM_CORES * NUM_SUBCORES
B_PER_W = BATCH // NUM_WORKERS
CHUNK = 128
N_CHUNKS = B_PER_W // CHUNK


def _make_kernel():
    mesh = plsc.VectorSubcoreMesh(core_axis_name="c", subcore_axis_name="s")

    @functools.partial(
        pl.kernel,
        mesh=mesh,
        out_type=jax.ShapeDtypeStruct((BATCH,), jnp.float32),
        compiler_params=pltpu.CompilerParams(use_tc_tiling_on_sc=False),
        scratch_types=[
            pltpu.VMEM((B_PER_W,), jnp.int32),
            pltpu.VMEM((B_PER_W,), jnp.int32),
            pltpu.VMEM((B_PER_W,), jnp.int32),
            pltpu.VMEM((B_PER_W,), jnp.int32),
            pltpu.VMEM((B_PER_W,), jnp.int32),
            pltpu.VMEM((B_PER_W,), jnp.int32),
            pltpu.VMEM((CHUNK, 2 * DIM), jnp.float32),
            pltpu.VMEM((CHUNK, 2 * DIM), jnp.float32),
            pltpu.VMEM((CHUNK, 2 * DIM), jnp.float32),
            pltpu.VMEM((B_PER_W,), jnp.float32),
            pltpu.SemaphoreType.DMA,
        ],
    )
    def distmult(head_hbm, rel_hbm, tail_hbm, ent_hbm, relemb_hbm, out_hbm,
                 hidx, ridx, tidx, hoff, roff, toff,
                 hrows, rrows, trows, scores, sem):
        wid = lax.axis_index("s") * NUM_CORES + lax.axis_index("c")
        base = wid * B_PER_W

        pltpu.sync_copy(head_hbm.at[pl.ds(base, B_PER_W)], hidx)
        pltpu.sync_copy(rel_hbm.at[pl.ds(base, B_PER_W)], ridx)
        pltpu.sync_copy(tail_hbm.at[pl.ds(base, B_PER_W)], tidx)

        def prep(v, carry):
            sl = pl.ds(v * LANES, LANES)
            for idx_ref, off_ref in ((hidx, hoff), (ridx, roff), (tidx, toff)):
                raw = idx_ref[sl]
                idx_ref[sl] = raw >> 1
                off_ref[sl] = (raw & 1) * DIM
            return carry

        lax.fori_loop(0, B_PER_W // LANES, prep, 0)

        lane = lax.iota(jnp.int32, LANES)
        dnums = lax.GatherDimensionNumbers(
            offset_dims=(), collapsed_slice_dims=(0,), start_index_map=(0,))

        def shuffle(v, idx):
            return lax.gather(v, idx[:, None], dnums, slice_sizes=(1,),
                              mode=lax.GatherScatterMode.PROMISE_IN_BOUNDS)

        def lane_sum(v):
            for sh in (8, 4, 2, 1):
                v = v + shuffle(v, lane ^ sh)
            return v

        for c in range(N_CHUNKS):
            csl = pl.ds(c * CHUNK, CHUNK)
            cps = [
                pltpu.async_copy(ent_hbm.at[hidx.at[csl]], hrows, sem),
                pltpu.async_copy(relemb_hbm.at[ridx.at[csl]], rrows, sem),
                pltpu.async_copy(ent_hbm.at[tidx.at[csl]], trows, sem),
            ]
            for cp in cps:
                cp.wait()

            def group(g, carry):
                svec = jnp.zeros((LANES,), jnp.float32)
                gsl = pl.ds(c * CHUNK + g * LANES, LANES)
                ohv = hoff[gsl]
                orv = roff[gsl]
                otv = toff[gsl]
                for j in range(LANES):
                    bb = g * LANES + j
                    oh = ohv[j]
                    orl = orv[j]
                    ot = otv[j]
                    acc = None
                    for dc in range(DIM // LANES):
                        d = dc * LANES
                        prod = (hrows[bb, pl.ds(oh + d, LANES)]
                                * rrows[bb, pl.ds(orl + d, LANES)]
                                * trows[bb, pl.ds(ot + d, LANES)])
                        acc = prod if acc is None else acc + prod
                    svec = jnp.where(lane == j, -lane_sum(acc), svec)
                scores[pl.ds(c * CHUNK + g * LANES, LANES)] = svec
                return carry

            lax.fori_loop(0, CHUNK // LANES, group, 0)

        pltpu.sync_copy(scores, out_hbm.at[pl.ds(base, B_PER_W)])

    return distmult


_distmult = _make_kernel()


def _transpose_kernel_body(src_ref, out_ref):
    ident = (lax.broadcasted_iota(jnp.int32, (DIM, DIM), 0)
             == lax.broadcasted_iota(jnp.int32, (DIM, DIM), 1)).astype(jnp.float32)
    out_ref[...] = lax.dot_general(
        src_ref[...], ident, (((0,), (0,)), ((), ())),
        preferred_element_type=jnp.float32)


def _transpose_table(table_t, blk):
    n = table_t.shape[1]
    grid = (n + blk - 1) // blk
    return pl.pallas_call(
        _transpose_kernel_body,
        grid=(grid,),
        in_specs=[pl.BlockSpec((DIM, blk), lambda j: (0, j))],
        out_specs=pl.BlockSpec((blk, DIM), lambda j: (j, 0)),
        out_shape=jax.ShapeDtypeStruct((n, DIM), jnp.float32),
    )(table_t)


@jax.jit
def kernel(head, rel, tail, entity_emb, relation_emb):
    ent_rows = _transpose_table(entity_emb.T, 8192)
    rel_rows = _transpose_table(relation_emb.T, 1024)
    ent2 = ent_rows.reshape(ent_rows.shape[0] // 2, 2 * DIM)
    rel2 = rel_rows.reshape(rel_rows.shape[0] // 2, 2 * DIM)
    return _distmult(head, rel, tail, ent2, rel2)


# speedup vs baseline: 1.0340x; 1.0340x over previous
import functools

import jax
import jax.numpy as jnp
from jax import lax
from jax.experimental import pallas as pl
from jax.experimental.pallas import tpu as pltpu
from jax.experimental.pallas import tpu_sc as plsc

BATCH = 16384
DIM = 64
LANES = 16
NUM_CORES = 2
NUM_SUBCORES = 16
NUM_WORKERS = NUM_CORES * NUM_SUBCORES
B_PER_W = BATCH // NUM_WORKERS
CHUNK = 128
N_CHUNKS = B_PER_W // CHUNK


def _make_kernel():
    mesh = plsc.VectorSubcoreMesh(core_axis_name="c", subcore_axis_name="s")

    @functools.partial(
        pl.kernel,
        mesh=mesh,
        out_type=jax.ShapeDtypeStruct((BATCH,), jnp.float32),
        compiler_params=pltpu.CompilerParams(use_tc_tiling_on_sc=False),
        scratch_types=[
            pltpu.VMEM((B_PER_W,), jnp.int32),
            pltpu.VMEM((B_PER_W,), jnp.int32),
            pltpu.VMEM((B_PER_W,), jnp.int32),
            pltpu.VMEM((B_PER_W,), jnp.int32),
            pltpu.VMEM((B_PER_W,), jnp.int32),
            pltpu.VMEM((B_PER_W,), jnp.int32),
            pltpu.VMEM((CHUNK, 2 * DIM), jnp.float32),
            pltpu.VMEM((CHUNK, 2 * DIM), jnp.float32),
            pltpu.VMEM((CHUNK, 2 * DIM), jnp.float32),
            pltpu.VMEM((B_PER_W,), jnp.float32),
            pltpu.SemaphoreType.DMA,
        ],
    )
    def distmult(head_hbm, rel_hbm, tail_hbm, ent_hbm, relemb_hbm, out_hbm,
                 hidx, ridx, tidx, hoff, roff, toff,
                 hrows, rrows, trows, scores, sem):
        wid = lax.axis_index("s") * NUM_CORES + lax.axis_index("c")
        base = wid * B_PER_W

        pltpu.sync_copy(head_hbm.at[pl.ds(base, B_PER_W)], hidx)
        pltpu.sync_copy(rel_hbm.at[pl.ds(base, B_PER_W)], ridx)
        pltpu.sync_copy(tail_hbm.at[pl.ds(base, B_PER_W)], tidx)

        def prep(v, carry):
            sl = pl.ds(v * LANES, LANES)
            for idx_ref, off_ref in ((hidx, hoff), (ridx, roff), (tidx, toff)):
                raw = idx_ref[sl]
                idx_ref[sl] = raw >> 1
                off_ref[sl] = (raw & 1) * DIM
            return carry

        lax.fori_loop(0, B_PER_W // LANES, prep, 0)

        lane = lax.iota(jnp.int32, LANES)
        dnums = lax.GatherDimensionNumbers(
            offset_dims=(), collapsed_slice_dims=(0,), start_index_map=(0,))

        def shuffle(v, idx):
            return lax.gather(v, idx[:, None], dnums, slice_sizes=(1,),
                              mode=lax.GatherScatterMode.PROMISE_IN_BOUNDS)

        def lane_sum(v):
            for sh in (8, 4, 2, 1):
                v = v + shuffle(v, lane ^ sh)
            return v

        for c in range(N_CHUNKS):
            csl = pl.ds(c * CHUNK, CHUNK)
            cps = [
                pltpu.async_copy(ent_hbm.at[hidx.at[csl]], hrows, sem),
                pltpu.async_copy(relemb_hbm.at[ridx.at[csl]], rrows, sem),
                pltpu.async_copy(ent_hbm.at[tidx.at[csl]], trows, sem),
            ]
            for cp in cps:
                cp.wait()

            def group(g, carry):
                svec = jnp.zeros((LANES,), jnp.float32)
                gsl = pl.ds(c * CHUNK + g * LANES, LANES)
                ohv = hoff[gsl]
                orv = roff[gsl]
                otv = toff[gsl]
                for j in range(LANES):
                    bb = g * LANES + j
                    oh = ohv[j]
                    orl = orv[j]
                    ot = otv[j]
                    acc = None
                    for dc in range(DIM // LANES):
                        d = dc * LANES
                        prod = (hrows[bb, pl.ds(oh + d, LANES)]
                                * rrows[bb, pl.ds(orl + d, LANES)]
                                * trows[bb, pl.ds(ot + d, LANES)])
                        acc = prod if acc is None else acc + prod
                    svec = jnp.where(lane == j, -lane_sum(acc), svec)
                scores[pl.ds(c * CHUNK + g * LANES, LANES)] = svec
                return carry

            lax.fori_loop(0, CHUNK // LANES, group, 0)

        pltpu.sync_copy(scores, out_hbm.at[pl.ds(base, B_PER_W)])

    return distmult


_distmult = _make_kernel()


def _transpose_kernel_body(src_ref, out_ref):
    ident = (lax.broadcasted_iota(jnp.int32, (DIM, DIM), 0)
             == lax.broadcasted_iota(jnp.int32, (DIM, DIM), 1)).astype(jnp.float32)
    out_ref[...] = lax.dot_general(
        src_ref[...], ident, (((0,), (0,)), ((), ())),
        preferred_element_type=jnp.float32)


def _transpose_table(table_t, blk):
    n = table_t.shape[1]
    grid = (n + blk - 1) // blk
    return pl.pallas_call(
        _transpose_kernel_body,
        grid=(grid,),
        in_specs=[pl.BlockSpec((DIM, blk), lambda j: (0, j))],
        out_specs=pl.BlockSpec((blk, DIM), lambda j: (j, 0)),
        out_shape=jax.ShapeDtypeStruct((n, DIM), jnp.float32),
    )(table_t)


@jax.jit
def kernel(head, rel, tail, entity_emb, relation_emb):
    ent_rows = _transpose_table(entity_emb.T, 16384)
    rel_rows = _transpose_table(relation_emb.T, 1024)
    ent2 = ent_rows.reshape(ent_rows.shape[0] // 2, 2 * DIM)
    rel2 = rel_rows.reshape(rel_rows.shape[0] // 2, 2 * DIM)
    return _distmult(head, rel, tail, ent2, rel2)


# speedup vs baseline: 2.6122x; 2.5262x over previous
import functools

import jax
import jax.numpy as jnp
from jax import lax
from jax.experimental import pallas as pl
from jax.experimental.pallas import tpu as pltpu
from jax.experimental.pallas import tpu_sc as plsc

BATCH = 16384
DIM = 64
LANES = 16
NUM_CORES = 2
NUM_SUBCORES = 16
NUM_WORKERS = NUM_CORES * NUM_SUBCORES
B_PER_W = BATCH // NUM_WORKERS
CHUNK = 128
N_CHUNKS = B_PER_W // CHUNK


def _make_kernel():
    mesh = plsc.VectorSubcoreMesh(core_axis_name="c", subcore_axis_name="s")

    @functools.partial(
        pl.kernel,
        mesh=mesh,
        out_type=jax.ShapeDtypeStruct((BATCH,), jnp.float32),
        compiler_params=pltpu.CompilerParams(use_tc_tiling_on_sc=False),
        scratch_types=[
            pltpu.VMEM((B_PER_W,), jnp.int32),
            pltpu.VMEM((B_PER_W,), jnp.int32),
            pltpu.VMEM((B_PER_W,), jnp.int32),
            pltpu.VMEM((B_PER_W,), jnp.int32),
            pltpu.VMEM((B_PER_W,), jnp.int32),
            pltpu.VMEM((B_PER_W,), jnp.int32),
            pltpu.VMEM((CHUNK, 2 * DIM), jnp.float32),
            pltpu.VMEM((CHUNK, 2 * DIM), jnp.float32),
            pltpu.VMEM((CHUNK, 2 * DIM), jnp.float32),
            pltpu.VMEM((B_PER_W,), jnp.float32),
            pltpu.SemaphoreType.DMA,
        ],
    )
    def distmult(head_hbm, rel_hbm, tail_hbm, ent_hbm, relemb_hbm, out_hbm,
                 hidx, ridx, tidx, hoff, roff, toff,
                 hrows, rrows, trows, scores, sem):
        wid = lax.axis_index("s") * NUM_CORES + lax.axis_index("c")
        base = wid * B_PER_W

        pltpu.sync_copy(head_hbm.at[pl.ds(base, B_PER_W)], hidx)
        pltpu.sync_copy(rel_hbm.at[pl.ds(base, B_PER_W)], ridx)
        pltpu.sync_copy(tail_hbm.at[pl.ds(base, B_PER_W)], tidx)

        def prep(v, carry):
            sl = pl.ds(v * LANES, LANES)
            for idx_ref, off_ref, s in ((hidx, hoff, 13), (ridx, roff, 9),
                                        (tidx, toff, 13)):
                raw = idx_ref[sl]
                idx_ref[sl] = ((raw >> (s + 1)) << s) | (raw & ((1 << s) - 1))
                off_ref[sl] = ((raw >> s) & 1) << 6
            return carry

        lax.fori_loop(0, B_PER_W // LANES, prep, 0)

        lane = lax.iota(jnp.int32, LANES)
        dnums = lax.GatherDimensionNumbers(
            offset_dims=(), collapsed_slice_dims=(0,), start_index_map=(0,))

        def shuffle(v, idx):
            return lax.gather(v, idx[:, None], dnums, slice_sizes=(1,),
                              mode=lax.GatherScatterMode.PROMISE_IN_BOUNDS)

        def lane_sum(v):
            for sh in (8, 4, 2, 1):
                v = v + shuffle(v, lane ^ sh)
            return v

        for c in range(N_CHUNKS):
            csl = pl.ds(c * CHUNK, CHUNK)
            cps = [
                pltpu.async_copy(ent_hbm.at[hidx.at[csl]], hrows, sem),
                pltpu.async_copy(relemb_hbm.at[ridx.at[csl]], rrows, sem),
                pltpu.async_copy(ent_hbm.at[tidx.at[csl]], trows, sem),
            ]
            for cp in cps:
                cp.wait()

            def group(g, carry):
                svec = jnp.zeros((LANES,), jnp.float32)
                gsl = pl.ds(c * CHUNK + g * LANES, LANES)
                ohv = hoff[gsl]
                orv = roff[gsl]
                otv = toff[gsl]
                for j in range(LANES):
                    bb = g * LANES + j
                    oh = ohv[j]
                    orl = orv[j]
                    ot = otv[j]
                    acc = None
                    for dc in range(DIM // LANES):
                        d = dc * LANES
                        prod = (hrows[bb, pl.ds(oh + d, LANES)]
                                * rrows[bb, pl.ds(orl + d, LANES)]
                                * trows[bb, pl.ds(ot + d, LANES)])
                        acc = prod if acc is None else acc + prod
                    svec = jnp.where(lane == j, -lane_sum(acc), svec)
                scores[pl.ds(c * CHUNK + g * LANES, LANES)] = svec
                return carry

            lax.fori_loop(0, CHUNK // LANES, group, 0)

        pltpu.sync_copy(scores, out_hbm.at[pl.ds(base, B_PER_W)])

    return distmult


_distmult = _make_kernel()


def _transpose_kernel_body(a_ref, b_ref, out_ref):
    ident = (lax.broadcasted_iota(jnp.int32, (DIM, DIM), 0)
             == lax.broadcasted_iota(jnp.int32, (DIM, DIM), 1)).astype(jnp.float32)
    dn = (((0,), (0,)), ((), ()))
    ta = lax.dot_general(a_ref[...], ident, dn, preferred_element_type=jnp.float32)
    tb = lax.dot_general(b_ref[...], ident, dn, preferred_element_type=jnp.float32)
    out_ref[...] = jnp.concatenate([ta, tb], axis=1)


def _transpose_table(table_t, half, grid):
    max_blk = (table_t.shape[1] - 1) // half
    return pl.pallas_call(
        _transpose_kernel_body,
        grid=(grid,),
        in_specs=[
            pl.BlockSpec((DIM, half), lambda j: (0, 2 * j)),
            pl.BlockSpec((DIM, half),
                         lambda j: (0, jnp.minimum(2 * j + 1, max_blk))),
        ],
        out_specs=pl.BlockSpec((half, 2 * DIM), lambda j: (j, 0)),
        out_shape=jax.ShapeDtypeStruct((grid * half, 2 * DIM), jnp.float32),
    )(table_t, table_t)


@jax.jit
def kernel(head, rel, tail, entity_emb, relation_emb):
    ent2 = _transpose_table(entity_emb.T, 8192, 62)
    rel2 = _transpose_table(relation_emb.T, 512, 1)
    return _distmult(head, rel, tail, ent2, rel2)


# speedup vs baseline: 3.3710x; 1.2905x over previous
import functools

import jax
import jax.numpy as jnp
from jax import lax
from jax.experimental import pallas as pl
from jax.experimental.pallas import tpu as pltpu
from jax.experimental.pallas import tpu_sc as plsc

BATCH = 16384
DIM = 64
LANES = 16
NUM_CORES = 2
NUM_SUBCORES = 16
NUM_WORKERS = NUM_CORES * NUM_SUBCORES
B_PER_W = BATCH // NUM_WORKERS
CHUNK = 128
N_CHUNKS = B_PER_W // CHUNK


def _make_kernel():
    mesh = plsc.VectorSubcoreMesh(core_axis_name="c", subcore_axis_name="s")

    @functools.partial(
        pl.kernel,
        mesh=mesh,
        out_type=jax.ShapeDtypeStruct((BATCH,), jnp.float32),
        compiler_params=pltpu.CompilerParams(use_tc_tiling_on_sc=False),
        scratch_types=[
            pltpu.VMEM((B_PER_W,), jnp.int32),
            pltpu.VMEM((B_PER_W,), jnp.int32),
            pltpu.VMEM((B_PER_W,), jnp.int32),
            pltpu.VMEM((B_PER_W,), jnp.int32),
            pltpu.VMEM((B_PER_W,), jnp.int32),
            pltpu.VMEM((B_PER_W,), jnp.int32),
            pltpu.VMEM((CHUNK, 2 * DIM), jnp.float32),
            pltpu.VMEM((CHUNK, 2 * DIM), jnp.float32),
            pltpu.VMEM((CHUNK, 2 * DIM), jnp.float32),
            pltpu.VMEM((B_PER_W,), jnp.float32),
            pltpu.SemaphoreType.DMA,
        ],
    )
    def distmult(head_hbm, rel_hbm, tail_hbm, ent_hbm, relemb_hbm, out_hbm,
                 hidx, ridx, tidx, hoff, roff, toff,
                 hrows, rrows, trows, scores, sem):
        wid = lax.axis_index("s") * NUM_CORES + lax.axis_index("c")
        base = wid * B_PER_W

        pltpu.sync_copy(head_hbm.at[pl.ds(base, B_PER_W)], hidx)
        pltpu.sync_copy(rel_hbm.at[pl.ds(base, B_PER_W)], ridx)
        pltpu.sync_copy(tail_hbm.at[pl.ds(base, B_PER_W)], tidx)

        def prep(v, carry):
            sl = pl.ds(v * LANES, LANES)
            for idx_ref, off_ref, s in ((hidx, hoff, 13), (ridx, roff, 9),
                                        (tidx, toff, 13)):
                raw = idx_ref[sl]
                idx_ref[sl] = ((raw >> (s + 1)) << s) | (raw & ((1 << s) - 1))
                off_ref[sl] = ((raw >> s) & 1) << 6
            return carry

        lax.fori_loop(0, B_PER_W // LANES, prep, 0)

        lane = lax.iota(jnp.int32, LANES)
        dnums = lax.GatherDimensionNumbers(
            offset_dims=(), collapsed_slice_dims=(0,), start_index_map=(0,))

        def shuffle(v, idx):
            return lax.gather(v, idx[:, None], dnums, slice_sizes=(1,),
                              mode=lax.GatherScatterMode.PROMISE_IN_BOUNDS)

        def lane_sum(v):
            for sh in (8, 4, 2, 1):
                v = v + shuffle(v, lane ^ sh)
            return v

        for c in range(N_CHUNKS):
            csl = pl.ds(c * CHUNK, CHUNK)
            cps = [
                pltpu.async_copy(ent_hbm.at[hidx.at[csl]], hrows, sem),
                pltpu.async_copy(relemb_hbm.at[ridx.at[csl]], rrows, sem),
                pltpu.async_copy(ent_hbm.at[tidx.at[csl]], trows, sem),
            ]
            for cp in cps:
                cp.wait()

            def group(g, carry):
                svec = jnp.zeros((LANES,), jnp.float32)
                gsl = pl.ds(c * CHUNK + g * LANES, LANES)
                ohv = hoff[gsl]
                orv = roff[gsl]
                otv = toff[gsl]
                for j in range(LANES):
                    bb = g * LANES + j
                    oh = ohv[j]
                    orl = orv[j]
                    ot = otv[j]
                    acc = None
                    for dc in range(DIM // LANES):
                        d = dc * LANES
                        prod = (hrows[bb, pl.ds(oh + d, LANES)]
                                * rrows[bb, pl.ds(orl + d, LANES)]
                                * trows[bb, pl.ds(ot + d, LANES)])
                        acc = prod if acc is None else acc + prod
                    svec = jnp.where(lane == j, -lane_sum(acc), svec)
                scores[pl.ds(c * CHUNK + g * LANES, LANES)] = svec
                return carry

            lax.fori_loop(0, CHUNK // LANES, group, 0)

        pltpu.sync_copy(scores, out_hbm.at[pl.ds(base, B_PER_W)])

    return distmult


_distmult = _make_kernel()


def _transpose_kernel_body(a_ref, b_ref, out_ref):
    ident = (lax.broadcasted_iota(jnp.int32, (2 * DIM, 2 * DIM), 0)
             == lax.broadcasted_iota(jnp.int32, (2 * DIM, 2 * DIM), 1)
             ).astype(jnp.float32)
    ab = jnp.concatenate([a_ref[...], b_ref[...]], axis=0)
    out_ref[...] = lax.dot_general(ab, ident, (((0,), (0,)), ((), ())),
                                   preferred_element_type=jnp.float32)


def _transpose_table(table_t, half, grid):
    max_blk = (table_t.shape[1] - 1) // half
    return pl.pallas_call(
        _transpose_kernel_body,
        grid=(grid,),
        in_specs=[
            pl.BlockSpec((DIM, half), lambda j: (0, 2 * j)),
            pl.BlockSpec((DIM, half),
                         lambda j: (0, jnp.minimum(2 * j + 1, max_blk))),
        ],
        out_specs=pl.BlockSpec((half, 2 * DIM), lambda j: (j, 0)),
        out_shape=jax.ShapeDtypeStruct((grid * half, 2 * DIM), jnp.float32),
    )(table_t, table_t)


@jax.jit
def kernel(head, rel, tail, entity_emb, relation_emb):
    ent2 = _transpose_table(entity_emb.T, 8192, 62)
    rel2 = _transpose_table(relation_emb.T, 512, 1)
    return _distmult(head, rel, tail, ent2, rel2)


# speedup vs baseline: 3.4473x; 1.0226x over previous
import functools

import jax
import jax.numpy as jnp
from jax import lax
from jax.experimental import pallas as pl
from jax.experimental.pallas import tpu as pltpu
from jax.experimental.pallas import tpu_sc as plsc

BATCH = 16384
DIM = 64
LANES = 16
NUM_CORES = 2
NUM_SUBCORES = 16
NUM_WORKERS = NUM_CORES * NUM_SUBCORES
B_PER_W = BATCH // NUM_WORKERS
CHUNK = 128
N_CHUNKS = B_PER_W // CHUNK
ENT_S = 14
ENT_GRID = 31
REL_S = 9


def _make_kernel():
    mesh = plsc.VectorSubcoreMesh(core_axis_name="c", subcore_axis_name="s")

    @functools.partial(
        pl.kernel,
        mesh=mesh,
        out_type=jax.ShapeDtypeStruct((BATCH,), jnp.float32),
        compiler_params=pltpu.CompilerParams(use_tc_tiling_on_sc=False),
        scratch_types=[
            pltpu.VMEM((B_PER_W,), jnp.int32),
            pltpu.VMEM((B_PER_W,), jnp.int32),
            pltpu.VMEM((B_PER_W,), jnp.int32),
            pltpu.VMEM((B_PER_W,), jnp.int32),
            pltpu.VMEM((B_PER_W,), jnp.int32),
            pltpu.VMEM((B_PER_W,), jnp.int32),
            pltpu.VMEM((CHUNK, 2 * DIM), jnp.float32),
            pltpu.VMEM((CHUNK, 2 * DIM), jnp.float32),
            pltpu.VMEM((CHUNK, 2 * DIM), jnp.float32),
            pltpu.VMEM((B_PER_W,), jnp.float32),
            pltpu.SemaphoreType.DMA,
        ],
    )
    def distmult(head_hbm, rel_hbm, tail_hbm, ent_hbm, relemb_hbm, out_hbm,
                 hidx, ridx, tidx, hoff, roff, toff,
                 hrows, rrows, trows, scores, sem):
        wid = lax.axis_index("s") * NUM_CORES + lax.axis_index("c")
        base = wid * B_PER_W

        pltpu.sync_copy(head_hbm.at[pl.ds(base, B_PER_W)], hidx)
        pltpu.sync_copy(rel_hbm.at[pl.ds(base, B_PER_W)], ridx)
        pltpu.sync_copy(tail_hbm.at[pl.ds(base, B_PER_W)], tidx)

        def prep(v, carry):
            sl = pl.ds(v * LANES, LANES)
            for idx_ref, off_ref, s in ((hidx, hoff, ENT_S), (ridx, roff, REL_S),
                                        (tidx, toff, ENT_S)):
                raw = idx_ref[sl]
                idx_ref[sl] = ((raw >> (s + 1)) << s) | (raw & ((1 << s) - 1))
                off_ref[sl] = ((raw >> s) & 1) << 6
            return carry

        lax.fori_loop(0, B_PER_W // LANES, prep, 0)

        lane = lax.iota(jnp.int32, LANES)
        dnums = lax.GatherDimensionNumbers(
            offset_dims=(), collapsed_slice_dims=(0,), start_index_map=(0,))

        def shuffle(v, idx):
            return lax.gather(v, idx[:, None], dnums, slice_sizes=(1,),
                              mode=lax.GatherScatterMode.PROMISE_IN_BOUNDS)

        def lane_sum(v):
            for sh in (8, 4, 2, 1):
                v = v + shuffle(v, lane ^ sh)
            return v

        for c in range(N_CHUNKS):
            csl = pl.ds(c * CHUNK, CHUNK)
            cps = [
                pltpu.async_copy(ent_hbm.at[hidx.at[csl]], hrows, sem),
                pltpu.async_copy(relemb_hbm.at[ridx.at[csl]], rrows, sem),
                pltpu.async_copy(ent_hbm.at[tidx.at[csl]], trows, sem),
            ]
            for cp in cps:
                cp.wait()

            def group(g, carry):
                svec = jnp.zeros((LANES,), jnp.float32)
                gsl = pl.ds(c * CHUNK + g * LANES, LANES)
                ohv = hoff[gsl]
                orv = roff[gsl]
                otv = toff[gsl]
                for j in range(LANES):
                    bb = g * LANES + j
                    oh = ohv[j]
                    orl = orv[j]
                    ot = otv[j]
                    acc = None
                    for dc in range(DIM // LANES):
                        d = dc * LANES
                        prod = (hrows[bb, pl.ds(oh + d, LANES)]
                                * rrows[bb, pl.ds(orl + d, LANES)]
                                * trows[bb, pl.ds(ot + d, LANES)])
                        acc = prod if acc is None else acc + prod
                    svec = jnp.where(lane == j, -lane_sum(acc), svec)
                scores[pl.ds(c * CHUNK + g * LANES, LANES)] = svec
                return carry

            lax.fori_loop(0, CHUNK // LANES, group, 0)

        pltpu.sync_copy(scores, out_hbm.at[pl.ds(base, B_PER_W)])

    return distmult


_distmult = _make_kernel()


def _transpose_kernel_body(a_ref, b_ref, out_ref):
    ident = (lax.broadcasted_iota(jnp.int32, (2 * DIM, 2 * DIM), 0)
             == lax.broadcasted_iota(jnp.int32, (2 * DIM, 2 * DIM), 1)
             ).astype(jnp.float32)
    ab = jnp.concatenate([a_ref[...], b_ref[...]], axis=0)
    out_ref[...] = lax.dot_general(ab, ident, (((0,), (0,)), ((), ())),
                                   preferred_element_type=jnp.float32)


def _transpose_table(table_t, half, grid):
    max_blk = (table_t.shape[1] - 1) // half
    return pl.pallas_call(
        _transpose_kernel_body,
        grid=(grid,),
        in_specs=[
            pl.BlockSpec((DIM, half), lambda j: (0, 2 * j)),
            pl.BlockSpec((DIM, half),
                         lambda j: (0, jnp.minimum(2 * j + 1, max_blk))),
        ],
        out_specs=pl.BlockSpec((half, 2 * DIM), lambda j: (j, 0)),
        out_shape=jax.ShapeDtypeStruct((grid * half, 2 * DIM), jnp.float32),
    )(table_t, table_t)


@jax.jit
def kernel(head, rel, tail, entity_emb, relation_emb):
    ent2 = _transpose_table(entity_emb.T, 1 << ENT_S, ENT_GRID)
    rel2 = _transpose_table(relation_emb.T, 1 << REL_S, 1)
    return _distmult(head, rel, tail, ent2, rel2)


# speedup vs baseline: 3.5456x; 1.0285x over previous
import functools

import jax
import jax.numpy as jnp
from jax import lax
from jax.experimental import pallas as pl
from jax.experimental.pallas import tpu as pltpu
from jax.experimental.pallas import tpu_sc as plsc

BATCH = 16384
DIM = 64
LANES = 16
NUM_CORES = 2
NUM_SUBCORES = 16
NUM_WORKERS = NUM_CORES * NUM_SUBCORES
B_PER_W = BATCH // NUM_WORKERS
CHUNK = 128
N_CHUNKS = B_PER_W // CHUNK
ENT_S = 14
ENT_GRID = 31
REL_S = 9


def _make_kernel():
    mesh = plsc.VectorSubcoreMesh(core_axis_name="c", subcore_axis_name="s")

    @functools.partial(
        pl.kernel,
        mesh=mesh,
        out_type=jax.ShapeDtypeStruct((BATCH,), jnp.float32),
        compiler_params=pltpu.CompilerParams(use_tc_tiling_on_sc=False),
        scratch_types=[
            pltpu.VMEM((B_PER_W,), jnp.int32),
            pltpu.VMEM((B_PER_W,), jnp.int32),
            pltpu.VMEM((B_PER_W,), jnp.int32),
            pltpu.VMEM((B_PER_W,), jnp.int32),
            pltpu.VMEM((B_PER_W,), jnp.int32),
            pltpu.VMEM((B_PER_W,), jnp.int32),
            pltpu.VMEM((CHUNK, 2 * DIM), jnp.float32),
            pltpu.VMEM((CHUNK, 2 * DIM), jnp.float32),
            pltpu.VMEM((CHUNK, 2 * DIM), jnp.float32),
            pltpu.VMEM((CHUNK, 2 * DIM), jnp.float32),
            pltpu.VMEM((CHUNK, 2 * DIM), jnp.float32),
            pltpu.VMEM((CHUNK, 2 * DIM), jnp.float32),
            pltpu.VMEM((B_PER_W,), jnp.float32),
            pltpu.SemaphoreType.DMA,
            pltpu.SemaphoreType.DMA,
        ],
    )
    def distmult(head_hbm, rel_hbm, tail_hbm, ent_hbm, relemb_hbm, out_hbm,
                 hidx, ridx, tidx, hoff, roff, toff,
                 hrows0, rrows0, trows0, hrows1, rrows1, trows1,
                 scores, sem0, sem1):
        wid = lax.axis_index("s") * NUM_CORES + lax.axis_index("c")
        base = wid * B_PER_W

        pltpu.sync_copy(head_hbm.at[pl.ds(base, B_PER_W)], hidx)
        pltpu.sync_copy(rel_hbm.at[pl.ds(base, B_PER_W)], ridx)
        pltpu.sync_copy(tail_hbm.at[pl.ds(base, B_PER_W)], tidx)

        def prep(v, carry):
            sl = pl.ds(v * LANES, LANES)
            for idx_ref, off_ref, s in ((hidx, hoff, ENT_S), (ridx, roff, REL_S),
                                        (tidx, toff, ENT_S)):
                raw = idx_ref[sl]
                idx_ref[sl] = ((raw >> (s + 1)) << s) | (raw & ((1 << s) - 1))
                off_ref[sl] = ((raw >> s) & 1) << 6
            return carry

        lax.fori_loop(0, B_PER_W // LANES, prep, 0)

        lane = lax.iota(jnp.int32, LANES)
        dnums = lax.GatherDimensionNumbers(
            offset_dims=(), collapsed_slice_dims=(0,), start_index_map=(0,))

        def shuffle(v, idx):
            return lax.gather(v, idx[:, None], dnums, slice_sizes=(1,),
                              mode=lax.GatherScatterMode.PROMISE_IN_BOUNDS)

        def lane_sum(v):
            for sh in (8, 4, 2, 1):
                v = v + shuffle(v, lane ^ sh)
            return v

        bufs = ((hrows0, rrows0, trows0, sem0), (hrows1, rrows1, trows1, sem1))

        def fire(c):
            hb, rb, tb, sem = bufs[c % 2]
            csl = pl.ds(c * CHUNK, CHUNK)
            return [
                pltpu.async_copy(ent_hbm.at[hidx.at[csl]], hb, sem),
                pltpu.async_copy(relemb_hbm.at[ridx.at[csl]], rb, sem),
                pltpu.async_copy(ent_hbm.at[tidx.at[csl]], tb, sem),
            ]

        cps = fire(0)
        for c in range(N_CHUNKS):
            nxt = fire(c + 1) if c + 1 < N_CHUNKS else []
            for cp in cps:
                cp.wait()
            cps = nxt
            hrows, rrows, trows, _ = bufs[c % 2]

            def group(g, carry):
                svec = jnp.zeros((LANES,), jnp.float32)
                gsl = pl.ds(c * CHUNK + g * LANES, LANES)
                ohv = hoff[gsl]
                orv = roff[gsl]
                otv = toff[gsl]
                for j in range(LANES):
                    bb = g * LANES + j
                    oh = ohv[j]
                    orl = orv[j]
                    ot = otv[j]
                    acc = None
                    for dc in range(DIM // LANES):
                        d = dc * LANES
                        prod = (hrows[bb, pl.ds(oh + d, LANES)]
                                * rrows[bb, pl.ds(orl + d, LANES)]
                                * trows[bb, pl.ds(ot + d, LANES)])
                        acc = prod if acc is None else acc + prod
                    svec = jnp.where(lane == j, -lane_sum(acc), svec)
                scores[pl.ds(c * CHUNK + g * LANES, LANES)] = svec
                return carry

            lax.fori_loop(0, CHUNK // LANES, group, 0)

        pltpu.sync_copy(scores, out_hbm.at[pl.ds(base, B_PER_W)])

    return distmult


_distmult = _make_kernel()


def _transpose_kernel_body(a_ref, b_ref, out_ref):
    ident = (lax.broadcasted_iota(jnp.int32, (2 * DIM, 2 * DIM), 0)
             == lax.broadcasted_iota(jnp.int32, (2 * DIM, 2 * DIM), 1)
             ).astype(jnp.float32)
    ab = jnp.concatenate([a_ref[...], b_ref[...]], axis=0)
    out_ref[...] = lax.dot_general(ab, ident, (((0,), (0,)), ((), ())),
                                   preferred_element_type=jnp.float32)


def _transpose_table(table_t, half, grid):
    max_blk = (table_t.shape[1] - 1) // half
    return pl.pallas_call(
        _transpose_kernel_body,
        grid=(grid,),
        in_specs=[
            pl.BlockSpec((DIM, half), lambda j: (0, 2 * j)),
            pl.BlockSpec((DIM, half),
                         lambda j: (0, jnp.minimum(2 * j + 1, max_blk))),
        ],
        out_specs=pl.BlockSpec((half, 2 * DIM), lambda j: (j, 0)),
        out_shape=jax.ShapeDtypeStruct((grid * half, 2 * DIM), jnp.float32),
    )(table_t, table_t)


@jax.jit
def kernel(head, rel, tail, entity_emb, relation_emb):
    ent2 = _transpose_table(entity_emb.T, 1 << ENT_S, ENT_GRID)
    rel2 = _transpose_table(relation_emb.T, 1 << REL_S, 1)
    return _distmult(head, rel, tail, ent2, rel2)
